# baseline (speedup 1.0000x reference)
import functools

import jax
import jax.numpy as jnp
from jax import lax
from jax.experimental import pallas as pl
from jax.experimental.pallas import tpu as pltpu
from jax.experimental.pallas import tpu_sc as plsc

_V = 10000
_E = 320000
_HID = 128
_OUT = 128

_NC = 2
_NS = 16
_NW = _NC * _NS
_EPS = _E // _NW
_CS = 40
_EPSP = 10000
_NST = 5
_SB = _EPSP // _NST // _CS
_RPS = _V // _NS
_ZR = 25
_FS = _HID // 16


def _sc_spmm_body(src0, dst0, val0, src1, dst1, val1, w0, w1, out,
                  acc, sidx, didx, vals, rows, zbuf):
    c = lax.axis_index("c")
    s = lax.axis_index("s")
    wid = c * _NS + s

    def _zrow(r, carry):
        for f in range(_FS):
            zbuf[r, pl.ds(f * 16, 16)] = jnp.zeros((16,), jnp.float32)
        return carry
    lax.fori_loop(0, _ZR, _zrow, 0)
    for j in range(_RPS // _ZR):
        pltpu.sync_copy(zbuf, acc.at[pl.ds(s * _RPS + j * _ZR, _ZR)])
    plsc.subcore_barrier()

    for (srcr, dstr, valr, wr) in ((src0, dst0, val0, w0),
                                   (src1, dst1, val1, w1)):
      for h in range(_NST):
        pltpu.sync_copy(srcr.at[wid, h], sidx)
        pltpu.sync_copy(dstr.at[wid, h], didx)
        pltpu.sync_copy(valr.at[wid, h], vals)

        def _chunk(ch, carry):
            pltpu.sync_copy(wr.at[sidx.at[ch]], rows)

            @plsc.parallel_loop(0, _CS, unroll=8)
            def _edge(i):
                vi = plsc.load_gather(
                    vals, [jnp.full((16,), ch, jnp.int32),
                           jnp.full((16,), i, jnp.int32)])
                for f in range(_FS):
                    sl = pl.ds(f * 16, 16)
                    rows[i, sl] = rows[i, sl] * vi

            pltpu.sync_copy(rows, acc.at[didx.at[ch]], add=True)
            return carry

        lax.fori_loop(0, _SB, _chunk, 0)

    plsc.subcore_barrier()
    pltpu.sync_copy(acc.at[pl.ds(s * _RPS, _RPS)],
                    out.at[c, pl.ds(s * _RPS, _RPS)])


@functools.lru_cache(maxsize=None)
def _make_sc_spmm():
  return functools.partial(
    pl.kernel,
    out_type=jax.ShapeDtypeStruct((_NC, _V, _HID), jnp.float32),
    mesh=plsc.VectorSubcoreMesh(core_axis_name="c", subcore_axis_name="s",
                                num_cores=_NC, num_subcores=_NS),
    scratch_types=[
        pltpu.VMEM_SHARED((_V, _HID), jnp.float32),
        pltpu.VMEM((_SB, _CS), jnp.int32),
        pltpu.VMEM((_SB, _CS), jnp.int32),
        pltpu.VMEM((_SB, _CS), jnp.float32),
        pltpu.VMEM((_CS, _HID), jnp.float32),
        pltpu.VMEM((_ZR, _HID), jnp.float32),
    ],
    compiler_params=pltpu.CompilerParams(use_tc_tiling_on_sc=False,
                                         needs_layout_passes=False),
  )(_sc_spmm_body)


_BD = 256


def _mm_body(x_ref, h_ref, w_ref, b_ref, o_ref):
    hs = h_ref[0] + h_ref[1]
    acc = jnp.dot(x_ref[...], hs, preferred_element_type=jnp.float32)
    o_ref[...] = lax.dot_general(
        acc, w_ref[...], (((1,), (1,)), ((), ())),
        preferred_element_type=jnp.float32) + b_ref[...]


_mm = pl.pallas_call(
    _mm_body,
    out_shape=jax.ShapeDtypeStruct((_BD, _OUT), jnp.float32),
)


def kernel(adj0_indices, adj0_values, adj1_indices, adj1_values, X_dv,
           W0, W1, fc_w, fc_b):
    B, D, V = X_dv.shape

    def _split(idx, vals):
        idx = idx.astype(jnp.int32)
        pad = ((0, 0), (0, _EPSP - _EPS))

        def _shape(a):
            a = jnp.pad(a.reshape(_NW, _EPS), pad)
            return a.reshape(_NW, _NST, _SB, _CS)
        return _shape(idx[1]), _shape(idx[0]), _shape(vals)

    s0, d0, v0 = _split(adj0_indices, adj0_values)
    s1, d1, v1 = _split(adj1_indices, adj1_values)

    hpart = _make_sc_spmm()(s0, d0, v0, s1, d1, v1, W0, W1)

    x2 = X_dv.reshape(B * D, V)
    out2 = _mm(x2, hpart, fc_w, fc_b.reshape(1, _OUT))
    return out2.reshape(B, D, _OUT)


# speedup vs baseline: 1.9431x; 1.9431x over previous
import functools

import jax
import jax.numpy as jnp
from jax import lax
from jax.experimental import pallas as pl
from jax.experimental.pallas import tpu as pltpu
from jax.experimental.pallas import tpu_sc as plsc

_V = 10000
_E = 320000
_HID = 128
_OUT = 128

_NC = 2
_NS = 16
_NW = _NC * _NS
_EPS = _E // _NW
_CS = 80
_EPSP = 10000
_NST = 5
_SB = _EPSP // _NST // _CS
_RPS = _V // _NS
_ZR = 25
_FS = _HID // 16


def _sc_spmm_body(src0, dst0, val0, src1, dst1, val1, w0, w1, out,
                  acc, sidx, didx, vals, rows, zbuf, gsem):
    c = lax.axis_index("c")
    s = lax.axis_index("s")
    wid = c * _NS + s

    def _zrow(r, carry):
        for f in range(_FS):
            zbuf[r, pl.ds(f * 16, 16)] = jnp.zeros((16,), jnp.float32)
        return carry
    lax.fori_loop(0, _ZR, _zrow, 0)
    for j in range(_RPS // _ZR):
        pltpu.sync_copy(zbuf, acc.at[pl.ds(s * _RPS + j * _ZR, _ZR)])
    plsc.subcore_barrier()

    for (srcr, dstr, valr, wr) in ((src0, dst0, val0, w0),
                                   (src1, dst1, val1, w1)):
      for h in range(_NST):
        pltpu.sync_copy(srcr.at[wid, h], sidx)
        pltpu.sync_copy(dstr.at[wid, h], didx)
        pltpu.sync_copy(valr.at[wid, h], vals)

        def g_issue(ch, b):
            pltpu.async_copy(wr.at[sidx.at[ch]], rows.at[b], gsem)

        def g_wait(ch, b):
            pltpu.make_async_copy(
                wr.at[sidx.at[ch]], rows.at[b], gsem).wait()

        def scale(ch, b):
            @plsc.parallel_loop(0, _CS, unroll=8)
            def _edge(i):
                vi = plsc.load_gather(
                    vals, [jnp.full((16,), ch, jnp.int32),
                           jnp.full((16,), i, jnp.int32)])
                for f in range(_FS):
                    sl = pl.ds(f * 16, 16)
                    rows[b, i, sl] = rows[b, i, sl] * vi

        def s_add(ch, b):
            pltpu.sync_copy(rows.at[b], acc.at[didx.at[ch]], add=True)

        g_issue(0, 0)

        def _pair(t, carry):
            for (off, b) in ((0, 0), (1, 1)):
                ch = 2 * t + off
                g_wait(ch, b)
                g_issue(ch + 1, 1 - b)
                scale(ch, b)
                s_add(ch, b)
            return carry
        lax.fori_loop(0, (_SB - 1) // 2, _pair, 0)

        g_wait(_SB - 1, 0)
        scale(_SB - 1, 0)
        s_add(_SB - 1, 0)

    plsc.subcore_barrier()
    pltpu.sync_copy(acc.at[pl.ds(s * _RPS, _RPS)],
                    out.at[c, pl.ds(s * _RPS, _RPS)])


@functools.lru_cache(maxsize=None)
def _make_sc_spmm():
  return functools.partial(
    pl.kernel,
    out_type=jax.ShapeDtypeStruct((_NC, _V, _HID), jnp.float32),
    mesh=plsc.VectorSubcoreMesh(core_axis_name="c", subcore_axis_name="s",
                                num_cores=_NC, num_subcores=_NS),
    scratch_types=[
        pltpu.VMEM_SHARED((_V, _HID), jnp.float32),
        pltpu.VMEM((_SB, _CS), jnp.int32),
        pltpu.VMEM((_SB, _CS), jnp.int32),
        pltpu.VMEM((_SB, _CS), jnp.float32),
        pltpu.VMEM((2, _CS, _HID), jnp.float32),
        pltpu.VMEM((_ZR, _HID), jnp.float32),
        pltpu.SemaphoreType.DMA,
    ],
    compiler_params=pltpu.CompilerParams(use_tc_tiling_on_sc=False,
                                         needs_layout_passes=False),
  )(_sc_spmm_body)


_BD = 256


def _mm_body(x_ref, h_ref, w_ref, b_ref, o_ref):
    hs = h_ref[0] + h_ref[1]
    acc = jnp.dot(x_ref[...], hs, preferred_element_type=jnp.float32)
    o_ref[...] = lax.dot_general(
        acc, w_ref[...], (((1,), (1,)), ((), ())),
        preferred_element_type=jnp.float32) + b_ref[...]


_mm = pl.pallas_call(
    _mm_body,
    out_shape=jax.ShapeDtypeStruct((_BD, _OUT), jnp.float32),
)


def kernel(adj0_indices, adj0_values, adj1_indices, adj1_values, X_dv,
           W0, W1, fc_w, fc_b):
    B, D, V = X_dv.shape

    def _split(idx, vals):
        idx = idx.astype(jnp.int32)
        pad = ((0, 0), (0, _EPSP - _EPS))

        def _shape(a):
            a = jnp.pad(a.reshape(_NW, _EPS), pad)
            return a.reshape(_NW, _NST, _SB, _CS)
        return _shape(idx[1]), _shape(idx[0]), _shape(vals)

    s0, d0, v0 = _split(adj0_indices, adj0_values)
    s1, d1, v1 = _split(adj1_indices, adj1_values)

    hpart = _make_sc_spmm()(s0, d0, v0, s1, d1, v1, W0, W1)

    x2 = X_dv.reshape(B * D, V)
    out2 = _mm(x2, hpart, fc_w, fc_b.reshape(1, _OUT))
    return out2.reshape(B, D, _OUT)


# speedup vs baseline: 2.0985x; 1.0800x over previous
import functools

import jax
import jax.numpy as jnp
from jax import lax
from jax.experimental import pallas as pl
from jax.experimental.pallas import tpu as pltpu
from jax.experimental.pallas import tpu_sc as plsc

_V = 10000
_E = 320000
_HID = 128
_OUT = 128

_NC = 2
_NS = 16
_NW = _NC * _NS
_EPS = 2 * _E // _NW
_CS = 80
_SB = 25
_NB = _EPS // (_SB * _CS)
_RPS = _V // _NS
_ZR = 25
_FS = _HID // 16


def _sc_spmm_body(src, dst, val, w, out,
                  acc, sidx, didx, vals, rows, zbuf, gsem, ssem0, ssem1):
    c = lax.axis_index("c")
    s = lax.axis_index("s")
    wid = c * _NS + s

    def _zrow(r, carry):
        for f in range(_FS):
            zbuf[r, pl.ds(f * 16, 16)] = jnp.zeros((16,), jnp.float32)
        return carry
    lax.fori_loop(0, _ZR, _zrow, 0)
    for j in range(_RPS // _ZR):
        pltpu.sync_copy(zbuf, acc.at[pl.ds(s * _RPS + j * _ZR, _ZR)])
    plsc.subcore_barrier()

    def _block(h, carry):
        pltpu.sync_copy(src.at[wid, h], sidx)
        pltpu.sync_copy(dst.at[wid, h], didx)
        pltpu.sync_copy(val.at[wid, h], vals)

        def g_issue(ch, b):
            pltpu.async_copy(w.at[sidx.at[ch]], rows.at[b], gsem)

        def g_wait(ch, b):
            pltpu.make_async_copy(
                w.at[sidx.at[ch]], rows.at[b], gsem).wait()

        def scale(ch, b):
            @plsc.parallel_loop(0, _CS, unroll=8)
            def _edge(i):
                vi = plsc.load_gather(
                    vals, [jnp.full((16,), ch, jnp.int32),
                           jnp.full((16,), i, jnp.int32)])
                for f in range(_FS):
                    sl = pl.ds(f * 16, 16)
                    rows[b, i, sl] = rows[b, i, sl] * vi

        def s_issue(ch, b, sem):
            pltpu.async_copy(rows.at[b], acc.at[didx.at[ch]], sem,
                             add=True)

        def s_wait(b, sem):
            pltpu.make_async_copy(
                rows.at[b], acc.at[didx.at[0]], sem).wait()

        sems = (ssem0, ssem1)

        def chunk(ch, wait_prev):
            b = ch % 4
            if wait_prev:
                s_wait((ch - 2) % 4, sems[ch % 2])
            if ch + 1 < _SB:
                g_issue(ch + 1, (ch + 1) % 4)
            g_wait(ch, b)
            scale(ch, b)
            s_issue(ch, b, sems[ch % 2])

        g_issue(0, 0)
        chunk(0, False)
        chunk(1, False)
        chunk(2, True)

        def _quad(t, carry):
            ch0 = 3 + 4 * t
            for off in range(4):
                ch = ch0 + off
                bb = (3 + off) % 4
                pp = (3 + off) % 2
                s_wait((bb + 2) % 4, sems[pp])
                g_issue(ch + 1, (bb + 1) % 4)
                g_wait(ch, bb)
                scale(ch, bb)
                s_issue(ch, bb, sems[pp])
            return carry
        lax.fori_loop(0, (_SB - 5) // 4, _quad, 0)

        chunk(_SB - 2, True)
        chunk(_SB - 1, True)
        s_wait((_SB - 2) % 4, sems[(_SB - 2) % 2])
        s_wait((_SB - 1) % 4, sems[(_SB - 1) % 2])
        return carry

    lax.fori_loop(0, _NB, _block, 0)

    plsc.subcore_barrier()
    pltpu.sync_copy(acc.at[pl.ds(s * _RPS, _RPS)],
                    out.at[c, pl.ds(s * _RPS, _RPS)])


@functools.lru_cache(maxsize=None)
def _make_sc_spmm():
  return functools.partial(
    pl.kernel,
    out_type=jax.ShapeDtypeStruct((_NC, _V, _HID), jnp.float32),
    mesh=plsc.VectorSubcoreMesh(core_axis_name="c", subcore_axis_name="s",
                                num_cores=_NC, num_subcores=_NS),
    scratch_types=[
        pltpu.VMEM_SHARED((_V, _HID), jnp.float32),
        pltpu.VMEM((_SB, _CS), jnp.int32),
        pltpu.VMEM((_SB, _CS), jnp.int32),
        pltpu.VMEM((_SB, _CS), jnp.float32),
        pltpu.VMEM((4, _CS, _HID), jnp.float32),
        pltpu.VMEM((_ZR, _HID), jnp.float32),
        pltpu.SemaphoreType.DMA,
        pltpu.SemaphoreType.DMA,
        pltpu.SemaphoreType.DMA,
    ],
    compiler_params=pltpu.CompilerParams(use_tc_tiling_on_sc=False,
                                         needs_layout_passes=False),
  )(_sc_spmm_body)


_BD = 256


def _mm_body(x_ref, h_ref, w_ref, b_ref, o_ref):
    hs = h_ref[0] + h_ref[1]
    acc = jnp.dot(x_ref[...], hs, preferred_element_type=jnp.float32)
    o_ref[...] = lax.dot_general(
        acc, w_ref[...], (((1,), (1,)), ((), ())),
        preferred_element_type=jnp.float32) + b_ref[...]


_mm = pl.pallas_call(
    _mm_body,
    out_shape=jax.ShapeDtypeStruct((_BD, _OUT), jnp.float32),
)


def kernel(adj0_indices, adj0_values, adj1_indices, adj1_values, X_dv,
           W0, W1, fc_w, fc_b):
    B, D, V = X_dv.shape

    i0 = adj0_indices.astype(jnp.int32)
    i1 = adj1_indices.astype(jnp.int32)
    epa = _EPS // 2

    def _shape(a0, a1):
        return jnp.concatenate(
            [a0.reshape(_NW, 1, epa), a1.reshape(_NW, 1, epa)],
            axis=1).reshape(_NW, _NB, _SB, _CS)

    src = _shape(i0[1], i1[1] + _V)
    dst = _shape(i0[0], i1[0])
    val = _shape(adj0_values, adj1_values)
    w = jnp.concatenate([W0, W1], axis=0)

    hpart = _make_sc_spmm()(src, dst, val, w)

    x2 = X_dv.reshape(B * D, V)
    out2 = _mm(x2, hpart, fc_w, fc_b.reshape(1, _OUT))
    return out2.reshape(B, D, _OUT)
